# baseline (speedup 1.0000x reference)
import jax
import jax.numpy as jnp
from jax import lax
from jax.experimental import pallas as pl
from jax.experimental.pallas import tpu as pltpu
from jax.experimental.pallas import tpu_sc as plsc

B = 16384
F = 26
V = 100000
D = 128
EPS = 1e-5

NC = 2
NS = 16
NW = NC * NS
RPW = B // NW
CHUNK = 16
NCHUNK = RPW // CHUNK
LG = D // 16
CROWS = F * CHUNK
SEGS = [(s, min(128, CROWS - s)) for s in range(0, CROWS, 128)]


def _tree_sum(vals):
    vals = list(vals)
    while len(vals) > 1:
        nxt = [vals[i] + vals[i + 1] for i in range(0, len(vals) - 1, 2)]
        if len(vals) % 2:
            nxt.append(vals[-1])
        vals = nxt
    return vals[0]


def _lane_sum(v):
    return jnp.full((16,), jnp.sum(v), dtype=jnp.float32)


def _rsqrt_nr(x16):
    i = plsc.bitcast(x16, jnp.int32)
    seed = jnp.full((16,), 0x5F3759DF, dtype=jnp.int32) - lax.shift_right_logical(i, 1)
    y = plsc.bitcast(seed, jnp.float32)
    for _ in range(2):
        y = y * (1.5 - 0.5 * x16 * y * y)
    return y


def _sc_body(tables_hbm, catx_hbm, gamma_hbm, beta_hbm, out_hbm,
             idx_all, buf0, buf1, outb0, gamma_v, beta_v,
             sem0, sem1, osem0):
    wid = lax.axis_index("s") * NC + lax.axis_index("c")
    base = wid * RPW

    pltpu.sync_copy(gamma_hbm, gamma_v)
    pltpu.sync_copy(beta_hbm, beta_v)

    pltpu.sync_copy(catx_hbm.at[wid], idx_all)

    @pl.loop(0, NCHUNK)
    def _offsets(c):
        for f in range(1, F):
            v = idx_all[c, pl.ds(f * CHUNK, 16)]
            idx_all[c, pl.ds(f * CHUNK, 16)] = v + f * V

    def fire(c, buf, sem):
        for s, n in SEGS:
            pltpu.async_copy(
                tables_hbm.at[idx_all.at[c, pl.ds(s, n)]],
                buf.at[pl.ds(s, n)], sem)

    def drain(buf, sem):
        for s, n in SEGS:
            pltpu.make_async_copy(tables_hbm.at[pl.ds(0, n)],
                                  buf.at[pl.ds(s, n)], sem).wait()

    def compute(c, buf, outb, osem, row0):
        @pl.loop(0, CHUNK, step=2)
        def _row(r0):
            for r in (r0, r0 + 1):
                acc = []
                for l in range(LG):
                    s = (buf[r, pl.ds(l * 16, 16)]
                         + buf[CHUNK + r, pl.ds(l * 16, 16)])
                    for f in range(2, F - 1, 2):
                        s = s + (buf[f * CHUNK + r, pl.ds(l * 16, 16)]
                                 + buf[(f + 1) * CHUNK + r, pl.ds(l * 16, 16)])
                    if F % 2:
                        s = s + buf[(F - 1) * CHUNK + r, pl.ds(l * 16, 16)]
                    acc.append(s)
                mean_v = _lane_sum(_tree_sum(acc)) * (1.0 / D)
                ex2_v = _lane_sum(_tree_sum([a * a for a in acc])) * (1.0 / D)
                dev = [acc[l] - mean_v for l in range(LG)]
                inv = _rsqrt_nr(ex2_v - mean_v * mean_v + EPS)
                for l in range(LG):
                    g = gamma_v[pl.ds(l * 16, 16)]
                    bta = beta_v[pl.ds(l * 16, 16)]
                    outb[r, pl.ds(l * 16, 16)] = dev[l] * inv * g + bta

        pltpu.async_copy(outb, out_hbm.at[pl.ds(row0 + c * CHUNK, CHUNK)], osem)

    def drain_out(outb, osem):
        pltpu.make_async_copy(outb, out_hbm.at[pl.ds(0, CHUNK)], osem).wait()

    fire(0, buf0, sem0)

    @pl.loop(0, NCHUNK, step=2)
    def _chunk(c):
        fire(c + 1, buf1, sem1)
        drain(buf0, sem0)

        @pl.when(c >= 1)
        def _():
            drain_out(outb0, osem0)

        compute(c, buf0, outb0, osem0, base)

        @pl.when(c + 2 < NCHUNK)
        def _():
            fire(c + 2, buf0, sem0)

        drain(buf1, sem1)
        drain_out(outb0, osem0)
        compute(c + 1, buf1, outb0, osem0, base)

    drain_out(outb0, osem0)


@jax.jit
def kernel(cat_x, tables, gamma, beta):
    tables_flat = tables.reshape(F * V, D)
    catx_r = (cat_x.reshape(NW, NCHUNK, CHUNK, F)
              .transpose(0, 1, 3, 2)
              .reshape(NW, NCHUNK, CROWS))

    mesh = plsc.VectorSubcoreMesh(core_axis_name="c", subcore_axis_name="s",
                                  num_cores=NC, num_subcores=NS)
    run = pl.kernel(
        _sc_body,
        out_type=jax.ShapeDtypeStruct((B, D), jnp.float32),
        mesh=mesh,
        compiler_params=pltpu.CompilerParams(needs_layout_passes=False),
        scratch_types=[
            pltpu.VMEM((NCHUNK, CROWS), jnp.int32),
            pltpu.VMEM((CROWS, D), jnp.float32),
            pltpu.VMEM((CROWS, D), jnp.float32),
            pltpu.VMEM((CHUNK, D), jnp.float32),
            pltpu.VMEM((D,), jnp.float32),
            pltpu.VMEM((D,), jnp.float32),
            pltpu.SemaphoreType.DMA,
            pltpu.SemaphoreType.DMA,
            pltpu.SemaphoreType.DMA,
        ],
    )
    return run(tables_flat, catx_r, gamma, beta)


# speedup vs baseline: 1.1074x; 1.1074x over previous
import jax
import jax.numpy as jnp
from jax import lax
from jax.experimental import pallas as pl
from jax.experimental.pallas import tpu as pltpu
from jax.experimental.pallas import tpu_sc as plsc

B = 16384
F = 26
V = 100000
D = 128
EPS = 1e-5

NC = 2
NS = 16
NW = NC * NS
RPW = B // NW
CHUNK = 16
NCHUNK = RPW // CHUNK
LG = D // 16


def _tree_sum(vals):
    vals = list(vals)
    while len(vals) > 1:
        nxt = [vals[i] + vals[i + 1] for i in range(0, len(vals) - 1, 2)]
        if len(vals) % 2:
            nxt.append(vals[-1])
        vals = nxt
    return vals[0]


def _lane_sum(v):
    return jnp.full((16,), jnp.sum(v), dtype=jnp.float32)


def _rsqrt_nr(x16):
    i = plsc.bitcast(x16, jnp.int32)
    seed = jnp.full((16,), 0x5F3759DF, dtype=jnp.int32) - lax.shift_right_logical(i, 1)
    y = plsc.bitcast(seed, jnp.float32)
    for _ in range(2):
        y = y * (1.5 - 0.5 * x16 * y * y)
    return y


def _sc_body(tables_hbm, catx_hbm, gamma_hbm, beta_hbm, out_hbm,
             idx_all, buf0, buf1, outb0, gamma_v, beta_v,
             sem0, sem1, osem0):
    wid = lax.axis_index("s") * NC + lax.axis_index("c")
    base = wid * RPW

    pltpu.sync_copy(gamma_hbm, gamma_v)
    pltpu.sync_copy(beta_hbm, beta_v)

    for f in range(F):
        pltpu.async_copy(catx_hbm.at[f, pl.ds(base, RPW)], idx_all.at[f], sem0)
    for f in range(F):
        pltpu.make_async_copy(catx_hbm.at[f, pl.ds(base, RPW)], idx_all.at[f],
                              sem0).wait()

    @pl.loop(0, RPW // 16)
    def _offsets(j):
        for f in range(F):
            v = idx_all[f, pl.ds(j * 16, 16)]
            idx_all[f, pl.ds(j * 16, 16)] = v + f * V

    def fire(c, buf, sem):
        for f in range(F):
            pltpu.async_copy(
                tables_hbm.at[idx_all.at[f, pl.ds(c * CHUNK, CHUNK)]],
                buf.at[f], sem)

    def drain(buf, sem):
        for f in range(F):
            pltpu.make_async_copy(tables_hbm.at[pl.ds(0, CHUNK)], buf.at[f],
                                  sem).wait()

    def compute(c, buf, outb, osem, row0):
        @pl.loop(0, CHUNK, step=2)
        def _row(r0):
            for r in (r0, r0 + 1):
                acc = []
                for l in range(LG):
                    s = buf[0, r, pl.ds(l * 16, 16)] + buf[1, r, pl.ds(l * 16, 16)]
                    for f in range(2, F - 1, 2):
                        s = s + (buf[f, r, pl.ds(l * 16, 16)]
                                 + buf[f + 1, r, pl.ds(l * 16, 16)])
                    if F % 2:
                        s = s + buf[F - 1, r, pl.ds(l * 16, 16)]
                    acc.append(s)
                mean_v = _lane_sum(_tree_sum(acc)) * (1.0 / D)
                ex2_v = _lane_sum(_tree_sum([a * a for a in acc])) * (1.0 / D)
                dev = [acc[l] - mean_v for l in range(LG)]
                inv = _rsqrt_nr(ex2_v - mean_v * mean_v + EPS)
                for l in range(LG):
                    g = gamma_v[pl.ds(l * 16, 16)]
                    bta = beta_v[pl.ds(l * 16, 16)]
                    outb[r, pl.ds(l * 16, 16)] = dev[l] * inv * g + bta

        pltpu.async_copy(outb, out_hbm.at[pl.ds(row0 + c * CHUNK, CHUNK)], osem)

    def drain_out(outb, osem):
        pltpu.make_async_copy(outb, out_hbm.at[pl.ds(0, CHUNK)], osem).wait()

    fire(0, buf0, sem0)

    @pl.loop(0, NCHUNK, step=2)
    def _chunk(c):
        fire(c + 1, buf1, sem1)
        drain(buf0, sem0)

        @pl.when(c >= 1)
        def _():
            drain_out(outb0, osem0)

        compute(c, buf0, outb0, osem0, base)

        @pl.when(c + 2 < NCHUNK)
        def _():
            fire(c + 2, buf0, sem0)

        drain(buf1, sem1)
        drain_out(outb0, osem0)
        compute(c + 1, buf1, outb0, osem0, base)

    drain_out(outb0, osem0)


@jax.jit
def kernel(cat_x, tables, gamma, beta):
    tables_flat = tables.reshape(F * V, D)
    catx_t = cat_x.T

    mesh = plsc.VectorSubcoreMesh(core_axis_name="c", subcore_axis_name="s",
                                  num_cores=NC, num_subcores=NS)
    run = pl.kernel(
        _sc_body,
        out_type=jax.ShapeDtypeStruct((B, D), jnp.float32),
        mesh=mesh,
        compiler_params=pltpu.CompilerParams(needs_layout_passes=False),
        scratch_types=[
            pltpu.VMEM((F, RPW), jnp.int32),
            pltpu.VMEM((F, CHUNK, D), jnp.float32),
            pltpu.VMEM((F, CHUNK, D), jnp.float32),
            pltpu.VMEM((CHUNK, D), jnp.float32),
            pltpu.VMEM((D,), jnp.float32),
            pltpu.VMEM((D,), jnp.float32),
            pltpu.SemaphoreType.DMA,
            pltpu.SemaphoreType.DMA,
            pltpu.SemaphoreType.DMA,
        ],
    )
    return run(tables_flat, catx_t, gamma, beta)
